# baseline (speedup 1.0000x reference)
import jax
import jax.numpy as jnp
from jax.experimental import pallas as pl
from jax.experimental.pallas import tpu as pltpu

T = 8192
D = 1024
G = 16
M = 16
K = 4
GM = G * M
BT = 1024
NEG = -1e30


def _router_kernel(h_ref, w_ref, fp_ref, gi_ref, aux_ref,
                   gp_acc, mp_acc):
    pid = pl.program_id(0)
    nb = pl.num_programs(0)

    h = h_ref[...]
    out = jnp.dot(h, w_ref[...], preferred_element_type=jnp.float32)
    ml = out[:, :GM]
    gl = out[:, GM:]

    gmax = jnp.max(gl, axis=-1, keepdims=True)
    ge = jnp.exp(gl - gmax)
    gs = jnp.sum(ge, axis=-1, keepdims=True)
    gp = ge / gs
    top_gp = jnp.max(gp, axis=-1, keepdims=True)
    giota = jax.lax.broadcasted_iota(jnp.int32, gl.shape, 1)
    gidx = jnp.min(jnp.where(gp == top_gp, giota, G), axis=-1,
                   keepdims=True)

    ciota = jax.lax.broadcasted_iota(jnp.int32, ml.shape, 1)
    mlz = jnp.where((ciota >> 4) == gidx, ml, 0.0)
    fr = jax.lax.broadcasted_iota(jnp.int32, (GM, M), 0) % M
    fc = jax.lax.broadcasted_iota(jnp.int32, (GM, M), 1)
    fold = (fr == fc).astype(jnp.float32)
    sel16 = jnp.dot(mlz, fold, preferred_element_type=jnp.float32)

    mmax = jnp.max(sel16, axis=-1, keepdims=True)
    me = jnp.exp(sel16 - mmax)
    ms = jnp.sum(me, axis=-1, keepdims=True)

    miota = jax.lax.broadcasted_iota(jnp.int32, me.shape, 1)
    bits = jax.lax.bitcast_convert_type(me, jnp.int32)
    packed = (bits & jnp.int32(-16)) | (jnp.int32(M - 1) - miota)
    vals = []
    idxs = []
    gbase = gidx * M
    for k in range(K):
        pmax = jnp.max(packed, axis=-1, keepdims=True)
        idxs.append(gbase + (jnp.int32(M - 1) - (pmax & jnp.int32(M - 1))))
        vals.append(jax.lax.bitcast_convert_type(
            pmax & jnp.int32(-16), jnp.float32))
        if k < K - 1:
            packed = jnp.where(packed == pmax,
                               jnp.int32(-2147483647), packed)

    topv = jnp.concatenate(vals, axis=1)
    fp_ref[...] = topv / jnp.sum(topv, axis=-1, keepdims=True)
    gi_ref[...] = jnp.concatenate(idxs, axis=1)

    ones_row = jnp.ones((1, BT), jnp.float32)
    gp_blk = jnp.dot(ones_row, gp, preferred_element_type=jnp.float32)
    mp_blk = jnp.dot(ones_row, me * (1.0 / ms),
                     preferred_element_type=jnp.float32)

    @pl.when(pid == 0)
    def _init():
        gp_acc[...] = gp_blk
        mp_acc[...] = mp_blk

    @pl.when(pid != 0)
    def _accum():
        gp_acc[...] += gp_blk
        mp_acc[...] += mp_blk

    @pl.when(pid == nb - 1)
    def _finalize():
        gmean = gp_acc[...] / T
        gloss = jnp.sum(gmean * gmean)
        mmean = mp_acc[...] / T
        mloss = jnp.sum(mmean * mmean)
        aux_ref[...] = jnp.reshape(gloss + mloss, (1, 1))


def kernel(hidden_states, group_gate_w, mini_gates):
    wm = jnp.transpose(mini_gates, (1, 0, 2)).reshape(D, GM)
    w = jnp.concatenate([wm, group_gate_w.T], axis=1)

    nb = T // BT
    fp, gi, aux = pl.pallas_call(
        _router_kernel,
        grid=(nb,),
        in_specs=[
            pl.BlockSpec((BT, D), lambda i: (i, 0)),
            pl.BlockSpec((D, GM + G), lambda i: (0, 0)),
        ],
        out_specs=[
            pl.BlockSpec((BT, K), lambda i: (i, 0)),
            pl.BlockSpec((BT, K), lambda i: (i, 0)),
            pl.BlockSpec((1, 1), lambda i: (0, 0)),
        ],
        out_shape=[
            jax.ShapeDtypeStruct((T, K), jnp.float32),
            jax.ShapeDtypeStruct((T, K), jnp.int32),
            jax.ShapeDtypeStruct((1, 1), jnp.float32),
        ],
        scratch_shapes=[
            pltpu.VMEM((1, G), jnp.float32),
            pltpu.VMEM((1, M), jnp.float32),
        ],
    )(hidden_states, w)

    return fp, gi, aux.reshape(())


# speedup vs baseline: 1.0136x; 1.0136x over previous
import jax
import jax.numpy as jnp
from jax.experimental import pallas as pl
from jax.experimental.pallas import tpu as pltpu

T = 8192
D = 1024
G = 16
M = 16
K = 4
GM = G * M
BT = 1024
NEG = -1e30


def _router_kernel(h_ref, w_ref, fp_ref, gi_ref, aux_ref,
                   gp_acc, mp_acc):
    pid = pl.program_id(0)
    nb = pl.num_programs(0)

    h = h_ref[...]
    out = jnp.dot(h, w_ref[...], preferred_element_type=jnp.float32)
    ml = out[:, :GM]
    gl = out[:, GM:]

    gmax = jnp.max(gl, axis=-1, keepdims=True)
    ge = jnp.exp(gl - gmax)
    gs = jnp.sum(ge, axis=-1, keepdims=True)
    gp = ge / gs
    top_gp = jnp.max(gp, axis=-1, keepdims=True)
    giota = jax.lax.broadcasted_iota(jnp.int32, gl.shape, 1)
    gidx = jnp.min(jnp.where(gp == top_gp, giota, G), axis=-1,
                   keepdims=True)

    ciota = jax.lax.broadcasted_iota(jnp.int32, ml.shape, 1)
    sel = (ciota >> 4) == gidx
    mlm = jnp.where(sel, ml, NEG)
    mmax = jnp.max(mlm, axis=-1, keepdims=True)
    me = jnp.exp(mlm - mmax)
    ms = jnp.sum(me, axis=-1, keepdims=True)

    bits = jax.lax.bitcast_convert_type(me, jnp.int32)
    packed = (bits & jnp.int32(-256)) | (jnp.int32(GM - 1) - ciota)
    packed = jnp.maximum(packed[:, :GM // 2], packed[:, GM // 2:])
    vals = []
    idxs = []
    for k in range(K):
        pmax = jnp.max(packed, axis=-1, keepdims=True)
        idxs.append(jnp.int32(GM - 1) - (pmax & jnp.int32(GM - 1)))
        vals.append(jax.lax.bitcast_convert_type(
            pmax & jnp.int32(-256), jnp.float32))
        if k < K - 1:
            packed = jnp.where(packed == pmax,
                               jnp.int32(-2147483647), packed)

    topv = jnp.concatenate(vals, axis=1)
    fp_ref[...] = topv / jnp.sum(topv, axis=-1, keepdims=True)
    gi_ref[...] = jnp.concatenate(idxs, axis=1)

    ones_row = jnp.ones((1, BT), jnp.float32)
    gp_blk = jnp.dot(ones_row, gp, preferred_element_type=jnp.float32)
    mp_blk = jnp.dot(ones_row, me * (1.0 / ms),
                     preferred_element_type=jnp.float32)

    @pl.when(pid == 0)
    def _init():
        gp_acc[...] = gp_blk
        mp_acc[...] = mp_blk

    @pl.when(pid != 0)
    def _accum():
        gp_acc[...] += gp_blk
        mp_acc[...] += mp_blk

    @pl.when(pid == nb - 1)
    def _finalize():
        gmean = gp_acc[...] / T
        gloss = jnp.sum(gmean * gmean)
        fr = jax.lax.broadcasted_iota(jnp.int32, (GM, M), 0) % M
        fc = jax.lax.broadcasted_iota(jnp.int32, (GM, M), 1)
        fold = (fr == fc).astype(jnp.float32)
        msum = jnp.dot(mp_acc[...], fold,
                       preferred_element_type=jnp.float32)
        mmean = msum / T
        mloss = jnp.sum(mmean * mmean)
        aux_ref[...] = jnp.reshape(gloss + mloss, (1, 1))


def kernel(hidden_states, group_gate_w, mini_gates):
    wm = jnp.transpose(mini_gates, (1, 0, 2)).reshape(D, GM)
    w = jnp.concatenate([wm, group_gate_w.T], axis=1)

    nb = T // BT
    fp, gi, aux = pl.pallas_call(
        _router_kernel,
        grid=(nb,),
        in_specs=[
            pl.BlockSpec((BT, D), lambda i: (i, 0)),
            pl.BlockSpec((D, GM + G), lambda i: (0, 0)),
        ],
        out_specs=[
            pl.BlockSpec((BT, K), lambda i: (i, 0)),
            pl.BlockSpec((BT, K), lambda i: (i, 0)),
            pl.BlockSpec((1, 1), lambda i: (0, 0)),
        ],
        out_shape=[
            jax.ShapeDtypeStruct((T, K), jnp.float32),
            jax.ShapeDtypeStruct((T, K), jnp.int32),
            jax.ShapeDtypeStruct((1, 1), jnp.float32),
        ],
        scratch_shapes=[
            pltpu.VMEM((1, G), jnp.float32),
            pltpu.VMEM((1, GM), jnp.float32),
        ],
    )(hidden_states, w)

    return fp, gi, aux.reshape(())


# speedup vs baseline: 1.0387x; 1.0248x over previous
import jax
import jax.numpy as jnp
from jax.experimental import pallas as pl
from jax.experimental.pallas import tpu as pltpu

T = 8192
D = 1024
G = 16
M = 16
K = 4
GM = G * M
BT = 1024
NEG = -1e30


def _router_kernel(h_ref, w_ref, fp_ref, gi_ref, aux_ref,
                   gp_acc, mp_acc):
    pid = pl.program_id(0)
    nb = pl.num_programs(0)

    h = h_ref[...]
    out = jnp.dot(h, w_ref[...], preferred_element_type=jnp.float32)
    ml = out[:, :GM]
    gl = out[:, GM:]

    gmax = jnp.max(gl, axis=-1, keepdims=True)
    ge = jnp.exp(gl - gmax)
    gs = jnp.sum(ge, axis=-1, keepdims=True)
    gp = ge / gs
    top_gp = jnp.max(gp, axis=-1, keepdims=True)
    giota = jax.lax.broadcasted_iota(jnp.int32, gl.shape, 1)
    gidx = jnp.min(jnp.where(gp == top_gp, giota, G), axis=-1,
                   keepdims=True)

    ciota = jax.lax.broadcasted_iota(jnp.int32, ml.shape, 1)
    sel = (ciota >> 4) == gidx
    mlm = jnp.where(sel, ml, NEG)
    mmax = jnp.max(mlm, axis=-1, keepdims=True)
    me = jnp.exp(mlm - mmax)
    ms = jnp.sum(me, axis=-1, keepdims=True)

    bits = jax.lax.bitcast_convert_type(me, jnp.int32)
    packed = (bits & jnp.int32(-256)) | (jnp.int32(GM - 1) - ciota)
    packed = jnp.maximum(packed[:, :GM // 2], packed[:, GM // 2:])
    vals = []
    idxs = []
    for k in range(K):
        pmax = jnp.max(packed, axis=-1, keepdims=True)
        idxs.append(jnp.int32(GM - 1) - (pmax & jnp.int32(GM - 1)))
        vals.append(jax.lax.bitcast_convert_type(
            pmax & jnp.int32(-256), jnp.float32))
        if k < K - 1:
            packed = jnp.where(packed == pmax,
                               jnp.int32(-2147483647), packed)

    topv = jnp.concatenate(vals, axis=1)
    fp_ref[...] = topv / jnp.sum(topv, axis=-1, keepdims=True)
    gi_ref[...] = jnp.concatenate(idxs, axis=1)

    gp_blk = jnp.sum(gp, axis=0, keepdims=True)
    mp_blk = jnp.sum(me * (1.0 / ms), axis=0, keepdims=True)

    @pl.when(pid == 0)
    def _init():
        gp_acc[...] = gp_blk
        mp_acc[...] = mp_blk

    @pl.when(pid != 0)
    def _accum():
        gp_acc[...] += gp_blk
        mp_acc[...] += mp_blk

    @pl.when(pid == nb - 1)
    def _finalize():
        gmean = gp_acc[...] / T
        gloss = jnp.sum(gmean * gmean)
        fr = jax.lax.broadcasted_iota(jnp.int32, (GM, M), 0) % M
        fc = jax.lax.broadcasted_iota(jnp.int32, (GM, M), 1)
        fold = (fr == fc).astype(jnp.float32)
        msum = jnp.dot(mp_acc[...], fold,
                       preferred_element_type=jnp.float32)
        mmean = msum / T
        mloss = jnp.sum(mmean * mmean)
        aux_ref[...] = jnp.reshape(gloss + mloss, (1, 1))


def kernel(hidden_states, group_gate_w, mini_gates):
    wm = jnp.transpose(mini_gates, (1, 0, 2)).reshape(D, GM)
    w = jnp.concatenate([wm, group_gate_w.T], axis=1)

    nb = T // BT
    fp, gi, aux = pl.pallas_call(
        _router_kernel,
        grid=(nb,),
        in_specs=[
            pl.BlockSpec((BT, D), lambda i: (i, 0)),
            pl.BlockSpec((D, GM + G), lambda i: (0, 0)),
        ],
        out_specs=[
            pl.BlockSpec((BT, K), lambda i: (i, 0)),
            pl.BlockSpec((BT, K), lambda i: (i, 0)),
            pl.BlockSpec((1, 1), lambda i: (0, 0)),
        ],
        out_shape=[
            jax.ShapeDtypeStruct((T, K), jnp.float32),
            jax.ShapeDtypeStruct((T, K), jnp.int32),
            jax.ShapeDtypeStruct((1, 1), jnp.float32),
        ],
        scratch_shapes=[
            pltpu.VMEM((1, G), jnp.float32),
            pltpu.VMEM((1, GM), jnp.float32),
        ],
    )(hidden_states, w)

    return fp, gi, aux.reshape(())


# speedup vs baseline: 1.3473x; 1.2971x over previous
import jax
import jax.numpy as jnp
from jax.experimental import pallas as pl
from jax.experimental.pallas import tpu as pltpu

T = 8192
D = 1024
G = 16
M = 16
K = 4
GM = G * M
BT = 1024
NEG = -1e30


def _router_kernel(h_ref, w_ref, fp_ref, gi_ref, aux_ref,
                   gp_acc, mp_acc):
    pid = pl.program_id(0)
    nb = pl.num_programs(0)

    h = h_ref[...]
    out = jnp.dot(h, w_ref[...], preferred_element_type=jnp.float32)
    mlt = jnp.transpose(out[:, :GM], (1, 0))
    glt = jnp.transpose(out[:, GM:], (1, 0))

    gmax = jnp.max(glt, axis=0, keepdims=True)
    ge = jnp.exp(glt - gmax)
    gs = jnp.sum(ge, axis=0, keepdims=True)
    gp = ge / gs
    top_gp = jnp.max(gp, axis=0, keepdims=True)
    giota = jax.lax.broadcasted_iota(jnp.int32, gp.shape, 0)
    gidx = jnp.min(jnp.where(gp == top_gp, giota, G), axis=0,
                   keepdims=True)

    riota = jax.lax.broadcasted_iota(jnp.int32, mlt.shape, 0)
    mlm = jnp.where((riota >> 4) == gidx, mlt, NEG)
    mmax = jnp.max(mlm, axis=0, keepdims=True)
    me = jnp.exp(mlm - mmax)
    ms = jnp.sum(me, axis=0, keepdims=True)

    bits = jax.lax.bitcast_convert_type(me, jnp.int32)
    packed = (bits & jnp.int32(-256)) | (jnp.int32(GM - 1) - riota)
    w = GM
    while w > M:
        w //= 2
        packed = jnp.maximum(packed[:w], packed[w:])
    vals = []
    idxs = []
    for k in range(K):
        pmax = jnp.max(packed, axis=0, keepdims=True)
        idxs.append(jnp.int32(GM - 1) - (pmax & jnp.int32(GM - 1)))
        vals.append(jax.lax.bitcast_convert_type(
            pmax & jnp.int32(-256), jnp.float32))
        if k < K - 1:
            packed = jnp.where(packed == pmax,
                               jnp.int32(-2147483647), packed)

    vals8 = jnp.concatenate(vals + vals, axis=0)
    idxs8 = jnp.concatenate(idxs + idxs, axis=0)
    fin8 = vals8 / jnp.sum(vals8[:K], axis=0, keepdims=True)
    fp_ref[...] = jnp.transpose(fin8, (1, 0))[:, :K]
    gi_ref[...] = jnp.transpose(idxs8, (1, 0))[:, :K]

    inv8 = jnp.broadcast_to(1.0 / ms, (8, BT))
    invcol = jnp.transpose(inv8, (1, 0))[:, :1]
    mp_blk = jnp.dot(me, invcol, preferred_element_type=jnp.float32)
    gp_blk = jnp.dot(gp, jnp.ones((BT, 1), jnp.float32),
                     preferred_element_type=jnp.float32)

    @pl.when(pid == 0)
    def _init():
        gp_acc[...] = gp_blk
        mp_acc[...] = mp_blk

    @pl.when(pid != 0)
    def _accum():
        gp_acc[...] += gp_blk
        mp_acc[...] += mp_blk

    @pl.when(pid == nb - 1)
    def _finalize():
        gmean = gp_acc[...] / T
        gloss = jnp.sum(gmean * gmean)
        a = mp_acc[...]
        wf = GM
        while wf > M:
            wf //= 2
            a = a[:wf] + a[wf:]
        mmean = a / T
        mloss = jnp.sum(mmean * mmean)
        aux_ref[...] = jnp.reshape(gloss + mloss, (1, 1))


def kernel(hidden_states, group_gate_w, mini_gates):
    wm = jnp.transpose(mini_gates, (1, 0, 2)).reshape(D, GM)
    w = jnp.concatenate([wm, group_gate_w.T], axis=1)

    nb = T // BT
    fp, gi, aux = pl.pallas_call(
        _router_kernel,
        grid=(nb,),
        in_specs=[
            pl.BlockSpec((BT, D), lambda i: (i, 0)),
            pl.BlockSpec((D, GM + G), lambda i: (0, 0)),
        ],
        out_specs=[
            pl.BlockSpec((BT, K), lambda i: (i, 0)),
            pl.BlockSpec((BT, K), lambda i: (i, 0)),
            pl.BlockSpec((1, 1), lambda i: (0, 0)),
        ],
        out_shape=[
            jax.ShapeDtypeStruct((T, K), jnp.float32),
            jax.ShapeDtypeStruct((T, K), jnp.int32),
            jax.ShapeDtypeStruct((1, 1), jnp.float32),
        ],
        scratch_shapes=[
            pltpu.VMEM((G, 1), jnp.float32),
            pltpu.VMEM((GM, 1), jnp.float32),
        ],
    )(hidden_states, w)

    return fp, gi, aux.reshape(())


# speedup vs baseline: 1.3527x; 1.0040x over previous
import jax
import jax.numpy as jnp
from jax.experimental import pallas as pl
from jax.experimental.pallas import tpu as pltpu

T = 8192
D = 1024
G = 16
M = 16
K = 4
GM = G * M
BT = 2048
NEG = -1e30


def _router_kernel(h_ref, w_ref, fp_ref, gi_ref, aux_ref,
                   gp_acc, mp_acc):
    pid = pl.program_id(0)
    nb = pl.num_programs(0)

    h = h_ref[...]
    out = jnp.dot(h, w_ref[...], preferred_element_type=jnp.float32)
    mlt = jnp.transpose(out[:, :GM], (1, 0))
    glt = jnp.transpose(out[:, GM:], (1, 0))

    gmax = jnp.max(glt, axis=0, keepdims=True)
    ge = jnp.exp(glt - gmax)
    gs = jnp.sum(ge, axis=0, keepdims=True)
    gp = ge / gs
    top_gp = jnp.max(gp, axis=0, keepdims=True)
    giota = jax.lax.broadcasted_iota(jnp.int32, gp.shape, 0)
    gidx = jnp.min(jnp.where(gp == top_gp, giota, G), axis=0,
                   keepdims=True)

    riota = jax.lax.broadcasted_iota(jnp.int32, mlt.shape, 0)
    mlm = jnp.where((riota >> 4) == gidx, mlt, NEG)
    mmax = jnp.max(mlm, axis=0, keepdims=True)
    me = jnp.exp(mlm - mmax)
    ms = jnp.sum(me, axis=0, keepdims=True)

    bits = jax.lax.bitcast_convert_type(me, jnp.int32)
    packed = (bits & jnp.int32(-256)) | (jnp.int32(GM - 1) - riota)
    w = GM
    while w > M:
        w //= 2
        packed = jnp.maximum(packed[:w], packed[w:])
    vals = []
    idxs = []
    for k in range(K):
        pmax = jnp.max(packed, axis=0, keepdims=True)
        idxs.append(jnp.int32(GM - 1) - (pmax & jnp.int32(GM - 1)))
        vals.append(jax.lax.bitcast_convert_type(
            pmax & jnp.int32(-256), jnp.float32))
        if k < K - 1:
            packed = jnp.where(packed == pmax,
                               jnp.int32(-2147483647), packed)

    vals8 = jnp.concatenate(vals + vals, axis=0)
    idxs8 = jnp.concatenate(idxs + idxs, axis=0)
    fin8 = vals8 / jnp.sum(vals8[:K], axis=0, keepdims=True)
    fp_ref[...] = jnp.transpose(fin8, (1, 0))[:, :K]
    gi_ref[...] = jnp.transpose(idxs8, (1, 0))[:, :K]

    inv8 = jnp.broadcast_to(1.0 / ms, (8, BT))
    invcol = jnp.transpose(inv8, (1, 0))[:, :1]
    mp_blk = jnp.dot(me, invcol, preferred_element_type=jnp.float32)
    gp_blk = jnp.dot(gp, jnp.ones((BT, 1), jnp.float32),
                     preferred_element_type=jnp.float32)

    @pl.when(pid == 0)
    def _init():
        gp_acc[...] = gp_blk
        mp_acc[...] = mp_blk

    @pl.when(pid != 0)
    def _accum():
        gp_acc[...] += gp_blk
        mp_acc[...] += mp_blk

    @pl.when(pid == nb - 1)
    def _finalize():
        gmean = gp_acc[...] / T
        gloss = jnp.sum(gmean * gmean)
        a = mp_acc[...]
        wf = GM
        while wf > M:
            wf //= 2
            a = a[:wf] + a[wf:]
        mmean = a / T
        mloss = jnp.sum(mmean * mmean)
        aux_ref[...] = jnp.reshape(gloss + mloss, (1, 1))


def kernel(hidden_states, group_gate_w, mini_gates):
    wm = jnp.transpose(mini_gates, (1, 0, 2)).reshape(D, GM)
    w = jnp.concatenate([wm, group_gate_w.T], axis=1)

    nb = T // BT
    fp, gi, aux = pl.pallas_call(
        _router_kernel,
        grid=(nb,),
        in_specs=[
            pl.BlockSpec((BT, D), lambda i: (i, 0)),
            pl.BlockSpec((D, GM + G), lambda i: (0, 0)),
        ],
        out_specs=[
            pl.BlockSpec((BT, K), lambda i: (i, 0)),
            pl.BlockSpec((BT, K), lambda i: (i, 0)),
            pl.BlockSpec((1, 1), lambda i: (0, 0)),
        ],
        out_shape=[
            jax.ShapeDtypeStruct((T, K), jnp.float32),
            jax.ShapeDtypeStruct((T, K), jnp.int32),
            jax.ShapeDtypeStruct((1, 1), jnp.float32),
        ],
        scratch_shapes=[
            pltpu.VMEM((G, 1), jnp.float32),
            pltpu.VMEM((GM, 1), jnp.float32),
        ],
    )(hidden_states, w)

    return fp, gi, aux.reshape(())
